# baseline (speedup 1.0000x reference)
import jax
import jax.numpy as jnp
from jax.experimental import pallas as pl
from jax.experimental.pallas import tpu as pltpu

_NB = 10


def _fused_mlp_kernel(x_ref, w1_ref, b1_ref, gamma_ref, beta_ref,
                      w2_ref, b2_ref, out_ref, h_scratch, stats_scratch):
    phase = pl.program_id(0)
    i = pl.program_id(1)
    rows = x_ref.shape[0]
    n_total = rows * pl.num_programs(1)

    @pl.when(phase == 0)
    def _matmul1_and_stats():
        xb = x_ref[...]
        hb = jax.lax.dot_general(
            xb, w1_ref[...], (((1,), (1,)), ((), ())),
            preferred_element_type=jnp.float32,
        ) + b1_ref[...]
        h_scratch[pl.ds(i * rows, rows), :] = hb
        s = jnp.sum(hb, axis=0, keepdims=True)
        sq = jnp.sum(hb * hb, axis=0, keepdims=True)

        @pl.when(i == 0)
        def _():
            stats_scratch[0:1, :] = s
            stats_scratch[1:2, :] = sq

        @pl.when(i > 0)
        def _():
            stats_scratch[0:1, :] = stats_scratch[0:1, :] + s
            stats_scratch[1:2, :] = stats_scratch[1:2, :] + sq

    @pl.when(phase == 1)
    def _normalize_matmul2_softmax():
        inv_n = 1.0 / n_total
        mean = stats_scratch[0:1, :] * inv_n
        var = stats_scratch[1:2, :] * inv_n - mean * mean
        scale = gamma_ref[...] * jax.lax.rsqrt(var + 1e-5)
        shift = beta_ref[...] - mean * scale
        hb = h_scratch[pl.ds(i * rows, rows), :]
        hn = jnp.maximum(hb * scale + shift, 0.0)
        logits = jax.lax.dot_general(
            hn, w2_ref[...], (((1,), (1,)), ((), ())),
            preferred_element_type=jnp.float32,
        ) + b2_ref[...]
        m = jnp.max(logits, axis=1, keepdims=True)
        shifted = logits - m
        lse = jnp.log(jnp.sum(jnp.exp(shifted), axis=1, keepdims=True))
        out_ref[...] = shifted - lse


def kernel(x, edge_index, W1, b1, gamma, beta, W2, b2):
    del edge_index
    n, feat = x.shape
    hid = W1.shape[0]
    nclass = W2.shape[0]
    rows = n // _NB

    def x_index(p, i):
        return (i + p * (_NB - 1 - i), 0)

    full = lambda p, i: (0, 0)
    return pl.pallas_call(
        _fused_mlp_kernel,
        grid=(2, _NB),
        in_specs=[
            pl.BlockSpec((rows, feat), x_index),
            pl.BlockSpec((hid, feat), full),
            pl.BlockSpec((1, hid), full),
            pl.BlockSpec((1, hid), full),
            pl.BlockSpec((1, hid), full),
            pl.BlockSpec((nclass, hid), full),
            pl.BlockSpec((1, nclass), full),
        ],
        out_specs=pl.BlockSpec((rows, nclass), lambda p, i: (i, 0)),
        out_shape=jax.ShapeDtypeStruct((n, nclass), jnp.float32),
        scratch_shapes=[
            pltpu.VMEM((n, hid), jnp.float32),
            pltpu.VMEM((8, hid), jnp.float32),
        ],
        compiler_params=pltpu.CompilerParams(
            dimension_semantics=("arbitrary", "arbitrary"),
        ),
    )(x, W1, b1.reshape(1, -1), gamma.reshape(1, -1), beta.reshape(1, -1),
      W2, b2.reshape(1, -1))


# speedup vs baseline: 1.3713x; 1.3713x over previous
import jax
import jax.numpy as jnp
from jax.experimental import pallas as pl


def _fused_mlp_kernel(x_ref, w1_ref, b1_ref, gamma_ref, beta_ref,
                      w2_ref, b2_ref, out_ref):
    x = x_ref[...]
    h = jax.lax.dot_general(
        x, w1_ref[...], (((1,), (1,)), ((), ())),
        preferred_element_type=jnp.float32,
    ) + b1_ref[...]

    n = h.shape[0]
    mean = jnp.sum(h, axis=0, keepdims=True) * (1.0 / n)
    centered = h - mean
    var = jnp.sum(centered * centered, axis=0, keepdims=True) * (1.0 / n)
    h = centered * jax.lax.rsqrt(var + 1e-5) * gamma_ref[...] + beta_ref[...]
    h = jnp.maximum(h, 0.0)

    logits = jax.lax.dot_general(
        h, w2_ref[...], (((1,), (1,)), ((), ())),
        preferred_element_type=jnp.float32,
    ) + b2_ref[...]

    m = jnp.max(logits, axis=1, keepdims=True)
    shifted = logits - m
    lse = jnp.log(jnp.sum(jnp.exp(shifted), axis=1, keepdims=True))
    out_ref[...] = shifted - lse


def kernel(x, edge_index, W1, b1, gamma, beta, W2, b2):
    del edge_index
    n = x.shape[0]
    nclass = W2.shape[0]
    return pl.pallas_call(
        _fused_mlp_kernel,
        out_shape=jax.ShapeDtypeStruct((n, nclass), jnp.float32),
    )(x, W1, b1, gamma, beta, W2, b2)


# speedup vs baseline: 2.6885x; 1.9605x over previous
import jax
import jax.numpy as jnp
from jax.experimental import pallas as pl


def _fused_mlp_kernel(x_ref, w1_ref, b1_ref, gamma_ref, beta_ref,
                      w2_ref, b2_ref, out_ref):
    x = x_ref[...]
    h = jax.lax.dot_general(
        x, w1_ref[...], (((1,), (1,)), ((), ())),
        preferred_element_type=jnp.float32,
    ) + b1_ref[...]

    n = h.shape[0]
    mean = jnp.sum(h, axis=0, keepdims=True) * (1.0 / n)
    centered = h - mean
    var = jnp.sum(centered * centered, axis=0, keepdims=True) * (1.0 / n)
    h = centered * jax.lax.rsqrt(var + 1e-5) * gamma_ref[...] + beta_ref[...]
    h = jnp.maximum(h, 0.0)

    logits_t = jax.lax.dot_general(
        w2_ref[...], h, (((1,), (1,)), ((), ())),
        preferred_element_type=jnp.float32,
    ) + b2_ref[...][:, None]

    m = jnp.max(logits_t, axis=0, keepdims=True)
    shifted = logits_t - m
    lse = jnp.log(jnp.sum(jnp.exp(shifted), axis=0, keepdims=True))
    out_ref[...] = shifted - lse


def kernel(x, edge_index, W1, b1, gamma, beta, W2, b2):
    del edge_index
    n = x.shape[0]
    nclass = W2.shape[0]
    out_t = pl.pallas_call(
        _fused_mlp_kernel,
        out_shape=jax.ShapeDtypeStruct((nclass, n), jnp.float32),
    )(x, W1, b1, gamma, beta, W2, b2)
    return out_t.T
